# baseline (speedup 1.0000x reference)
import functools

import jax
import jax.numpy as jnp
from jax import lax
from jax.experimental import pallas as pl
from jax.experimental.pallas import tpu as pltpu
from jax.experimental.pallas import tpu_sc as plsc

NUM_CORES = 2
NUM_SUBCORES = 16
NUM_WORKERS = NUM_CORES * NUM_SUBCORES
LANES = 16

B = 4096
S = 200
D = 64
BBLK = 256
ITEMS_PER_S = B // BBLK
N_ITEMS = S * ITEMS_PER_S
PER_W = N_ITEMS // NUM_WORKERS
R8 = D // 8
BT = BBLK // 128


def _emb_body(x_hbm, pos_hbm, table_hbm, out_hbm,
              idx_v, rows_v, pos_v, isem, g_sem, p_sem, o_sem):
    wid = lax.axis_index("s") * NUM_CORES + lax.axis_index("c")
    gbase = wid * PER_W
    iota = lax.iota(jnp.int32, LANES)

    def item_coords(g):
        s = g // ITEMS_PER_S
        bt2 = g % ITEMS_PER_S
        return s, bt2 * BBLK

    def start_fetch(g, bslot):
        s, b0 = item_coords(g)
        pltpu.sync_copy(x_hbm.at[s, pl.ds(b0, BBLK)], idx_v.at[bslot])
        pltpu.async_copy(table_hbm.at[idx_v.at[bslot]], rows_v.at[bslot],
                         g_sem.at[bslot])
        pltpu.async_copy(
            pos_hbm.at[pl.ds(s * R8, R8), pl.ds(b0 // 128, BT)],
            pos_v.at[bslot], p_sem.at[bslot])

    def wait_fetch(bslot):
        pltpu.make_async_copy(table_hbm.at[idx_v.at[bslot]],
                              rows_v.at[bslot], g_sem.at[bslot]).wait()
        pltpu.make_async_copy(pos_hbm.at[pl.ds(0, R8), pl.ds(0, BT)],
                              pos_v.at[bslot], p_sem.at[bslot]).wait()

    def wait_out(bslot):
        pltpu.make_async_copy(pos_v.at[bslot],
                              out_hbm.at[pl.ds(0, R8), pl.ds(0, BT)],
                              o_sem.at[bslot]).wait()

    start_fetch(gbase, 0)

    def pair_body(p, carry):
        for bslot in range(2):
            g = gbase + 2 * p + bslot
            wait_fetch(bslot)
            nxt = 1 - bslot

            @pl.when(2 * p + bslot + 1 < PER_W)
            def _():
                @pl.when(2 * p + bslot >= 1)
                def _():
                    wait_out(nxt)

                start_fetch(g + 1, nxt)

            def d_body(d, c):
                dcol = jnp.full((LANES,), d, dtype=jnp.int32)
                d8 = d // 8
                dm8 = d % 8
                for l16 in range(BBLK // LANES):
                    brow = iota + (l16 * LANES)
                    v = plsc.load_gather(rows_v.at[bslot], [brow, dcol])
                    sl = pl.ds((l16 % 8) * LANES, LANES)
                    bt = l16 // 8
                    pos_v[bslot, d8, bt, dm8, sl] = (
                        pos_v[bslot, d8, bt, dm8, sl] + v)
                return c

            lax.fori_loop(0, D, d_body, 0)
            s, b0 = item_coords(g)
            pltpu.async_copy(
                pos_v.at[bslot],
                out_hbm.at[pl.ds(s * R8, R8), pl.ds(b0 // 128, BT)],
                o_sem.at[bslot])
        return carry

    lax.fori_loop(0, PER_W // 2, pair_body, 0)
    wait_out(0)
    wait_out(1)


def kernel(x, positional_embedding, table):
    n_r8 = S * R8
    nbt = B // 128

    xt = x.T
    pos_lin = (positional_embedding.transpose(1, 2, 0)
               .reshape(n_r8, 8, nbt, 128)
               .transpose(0, 2, 1, 3))

    mesh = plsc.VectorSubcoreMesh(
        core_axis_name="c", subcore_axis_name="s",
        num_cores=NUM_CORES, num_subcores=NUM_SUBCORES)

    emb = functools.partial(
        pl.kernel,
        out_type=jax.ShapeDtypeStruct((n_r8, nbt, 8, 128), jnp.float32),
        mesh=mesh,
        scratch_types=[
            pltpu.VMEM((2, BBLK), jnp.int32),
            pltpu.VMEM((2, BBLK, D), jnp.float32),
            pltpu.VMEM((2, R8, BT, 8, 128), jnp.float32),
            pltpu.SemaphoreType.DMA,
            pltpu.SemaphoreType.DMA((2,)),
            pltpu.SemaphoreType.DMA((2,)),
            pltpu.SemaphoreType.DMA((2,)),
        ],
        compiler_params=pltpu.CompilerParams(
            use_tc_tiling_on_sc=False, needs_layout_passes=False),
    )(_emb_body)

    out = emb(xt, pos_lin, table)
    return (out.transpose(0, 2, 1, 3)
            .reshape(S, D, B)
            .transpose(2, 0, 1))


# speedup vs baseline: 1.0738x; 1.0738x over previous
import functools

import jax
import jax.numpy as jnp
from jax import lax
from jax.experimental import pallas as pl
from jax.experimental.pallas import tpu as pltpu
from jax.experimental.pallas import tpu_sc as plsc

NUM_CORES = 2
NUM_SUBCORES = 16
NUM_WORKERS = NUM_CORES * NUM_SUBCORES
LANES = 16

B = 4096
S = 200
D = 64
BBLK = 256
ITEMS_PER_S = B // BBLK
N_ITEMS = S * ITEMS_PER_S
PER_W = N_ITEMS // NUM_WORKERS
CHUNK_W = 1024
CHUNKS = 16
ITEM_W = CHUNK_W * CHUNKS
NBT = B // 128


def _emb_body(x_hbm, pos_hbm, table_hbm, out_hbm,
              idx_v, rows_v, pos_v, g_sem, p_sem, o_sem):
    wid = lax.axis_index("s") * NUM_CORES + lax.axis_index("c")
    gbase = wid * PER_W
    iota = lax.iota(jnp.int32, LANES)
    pat = (iota // 8) * 2048 + (iota % 8) * 128

    def item_coords(g):
        s = g // ITEMS_PER_S
        bt2 = g % ITEMS_PER_S
        return s, bt2 * BBLK

    def chunk_off(g, j):
        s, b0 = item_coords(g)
        return ((s * 8 + j // 2) * NBT + (b0 // 128 + j % 2)) * CHUNK_W

    def start_fetch(g, bslot):
        s, b0 = item_coords(g)
        pltpu.sync_copy(x_hbm.at[s, pl.ds(b0, BBLK)], idx_v.at[bslot])
        pltpu.async_copy(table_hbm.at[idx_v.at[bslot]], rows_v.at[bslot],
                         g_sem.at[bslot])
        for j in range(CHUNKS):
            pltpu.async_copy(pos_hbm.at[pl.ds(chunk_off(g, j), CHUNK_W)],
                             pos_v.at[bslot, pl.ds(j * CHUNK_W, CHUNK_W)],
                             p_sem.at[bslot])

    def wait_fetch(bslot):
        pltpu.make_async_copy(table_hbm.at[idx_v.at[bslot]],
                              rows_v.at[bslot], g_sem.at[bslot]).wait()
        pltpu.make_async_copy(pos_hbm.at[pl.ds(0, ITEM_W)],
                              pos_v.at[bslot], p_sem.at[bslot]).wait()

    def wait_out(bslot):
        pltpu.make_async_copy(pos_v.at[bslot], out_hbm.at[pl.ds(0, ITEM_W)],
                              o_sem.at[bslot]).wait()

    start_fetch(gbase, 0)

    def pair_body(p, carry):
        for bslot in range(2):
            g = gbase + 2 * p + bslot
            wait_fetch(bslot)
            nxt = 1 - bslot

            @pl.when(2 * p + bslot + 1 < PER_W)
            def _():
                @pl.when(2 * p + bslot >= 1)
                def _():
                    wait_out(nxt)

                start_fetch(g + 1, nxt)

            def b_body(b, c):
                bt = b // 128
                bl = b - bt * 128
                sb = bt * CHUNK_W + bl
                for k in range(D // LANES):
                    v = rows_v[bslot, b, pl.ds(k * LANES, LANES)]
                    offs = pat + (sb + 4096 * k)
                    plsc.addupdate_scatter(pos_v.at[bslot], [offs], v)
                return c

            lax.fori_loop(0, BBLK, b_body, 0, unroll=4)
            for j in range(CHUNKS):
                pltpu.async_copy(pos_v.at[bslot, pl.ds(j * CHUNK_W, CHUNK_W)],
                                 out_hbm.at[pl.ds(chunk_off(g, j), CHUNK_W)],
                                 o_sem.at[bslot])
        return carry

    lax.fori_loop(0, PER_W // 2, pair_body, 0)
    wait_out(0)
    wait_out(1)


def kernel(x, positional_embedding, table):
    xt = x.T
    pos_lin = (positional_embedding.transpose(1, 2, 0)
               .reshape(S * 8, 8, NBT, 128)
               .transpose(0, 2, 1, 3)
               .reshape(-1))

    mesh = plsc.VectorSubcoreMesh(
        core_axis_name="c", subcore_axis_name="s",
        num_cores=NUM_CORES, num_subcores=NUM_SUBCORES)

    emb = functools.partial(
        pl.kernel,
        out_type=jax.ShapeDtypeStruct((B * S * D,), jnp.float32),
        mesh=mesh,
        scratch_types=[
            pltpu.VMEM((2, BBLK), jnp.int32),
            pltpu.VMEM((2, BBLK, D), jnp.float32),
            pltpu.VMEM((2, ITEM_W), jnp.float32),
            pltpu.SemaphoreType.DMA((2,)),
            pltpu.SemaphoreType.DMA((2,)),
            pltpu.SemaphoreType.DMA((2,)),
        ],
        compiler_params=pltpu.CompilerParams(
            use_tc_tiling_on_sc=False, needs_layout_passes=False),
    )(_emb_body)

    out = emb(xt, pos_lin, table)
    return (out.reshape(S * 8, NBT, 8, 128)
            .transpose(0, 2, 1, 3)
            .reshape(S, D, B)
            .transpose(2, 0, 1))


# speedup vs baseline: 1.1635x; 1.0836x over previous
import functools

import jax
import jax.numpy as jnp
from jax import lax
from jax.experimental import pallas as pl
from jax.experimental.pallas import tpu as pltpu
from jax.experimental.pallas import tpu_sc as plsc

NUM_CORES = 2
NUM_SUBCORES = 16
NUM_WORKERS = NUM_CORES * NUM_SUBCORES
LANES = 16

B = 4096
S = 200
D = 64
BBLK = 256
ITEMS_PER_S = B // BBLK
N_ITEMS = S * ITEMS_PER_S
PER_W = N_ITEMS // NUM_WORKERS
CHUNK_W = 1024
CHUNKS = 16
ITEM_W = CHUNK_W * CHUNKS
NBT = B // 128


def _emb_body(x_hbm, pos_hbm, table_hbm, out_hbm,
              idx_v, rows_v, pos_v, g_sem, p_sem, o_sem):
    wid = lax.axis_index("s") * NUM_CORES + lax.axis_index("c")
    gbase = wid * PER_W
    iota = lax.iota(jnp.int32, LANES)
    pat = (iota // 8) * 2048 + (iota % 8) * 128

    def item_coords(g):
        s = g // ITEMS_PER_S
        bt2 = g % ITEMS_PER_S
        return s, bt2 * BBLK

    def chunk_off(g, j):
        s, b0 = item_coords(g)
        return ((s * 8 + j // 2) * NBT + (b0 // 128 + j % 2)) * CHUNK_W

    def start_fetch(g, bslot):
        s, b0 = item_coords(g)
        pltpu.sync_copy(x_hbm.at[s, pl.ds(b0, BBLK)], idx_v.at[bslot])
        pltpu.async_copy(table_hbm.at[idx_v.at[bslot]], rows_v.at[bslot],
                         g_sem.at[bslot])
        for j in range(CHUNKS):
            pltpu.async_copy(pos_hbm.at[pl.ds(chunk_off(g, j), CHUNK_W)],
                             pos_v.at[bslot, pl.ds(j * CHUNK_W, CHUNK_W)],
                             p_sem.at[bslot])

    def wait_fetch(bslot):
        pltpu.make_async_copy(table_hbm.at[idx_v.at[bslot]],
                              rows_v.at[bslot], g_sem.at[bslot]).wait()
        pltpu.make_async_copy(pos_hbm.at[pl.ds(0, ITEM_W)],
                              pos_v.at[bslot], p_sem.at[bslot]).wait()

    def wait_out(bslot):
        pltpu.make_async_copy(pos_v.at[bslot], out_hbm.at[pl.ds(0, ITEM_W)],
                              o_sem.at[bslot]).wait()

    start_fetch(gbase, 0)

    def pair_body(p, carry):
        for bslot in range(2):
            g = gbase + 2 * p + bslot
            wait_fetch(bslot)
            nxt = 1 - bslot

            @pl.when(2 * p + bslot + 1 < PER_W)
            def _():
                @pl.when(2 * p + bslot >= 1)
                def _():
                    wait_out(nxt)

                start_fetch(g + 1, nxt)

            def b_body(b, c):
                bt = b // 128
                bl = b - bt * 128
                sb = bt * CHUNK_W + bl
                vals = [rows_v[bslot, b, pl.ds(k * LANES, LANES)]
                        for k in range(D // LANES)]
                offs = [pat + (sb + 4096 * k) for k in range(D // LANES)]
                for k in range(D // LANES):
                    plsc.addupdate_scatter(pos_v.at[bslot], [offs[k]],
                                           vals[k])
                return c

            lax.fori_loop(0, BBLK, b_body, 0, unroll=4)
            for j in range(CHUNKS):
                pltpu.async_copy(pos_v.at[bslot, pl.ds(j * CHUNK_W, CHUNK_W)],
                                 out_hbm.at[pl.ds(chunk_off(g, j), CHUNK_W)],
                                 o_sem.at[bslot])
        return carry

    lax.fori_loop(0, PER_W // 2, pair_body, 0)
    wait_out(0)
    wait_out(1)


def kernel(x, positional_embedding, table):
    xt = x.T
    pos_lin = (positional_embedding.transpose(1, 2, 0)
               .reshape(S * 8, 8, NBT, 128)
               .transpose(0, 2, 1, 3)
               .reshape(-1))

    mesh = plsc.VectorSubcoreMesh(
        core_axis_name="c", subcore_axis_name="s",
        num_cores=NUM_CORES, num_subcores=NUM_SUBCORES)

    emb = functools.partial(
        pl.kernel,
        out_type=jax.ShapeDtypeStruct((B * S * D,), jnp.float32),
        mesh=mesh,
        scratch_types=[
            pltpu.VMEM((2, BBLK), jnp.int32),
            pltpu.VMEM((2, BBLK, D), jnp.float32),
            pltpu.VMEM((2, ITEM_W), jnp.float32),
            pltpu.SemaphoreType.DMA((2,)),
            pltpu.SemaphoreType.DMA((2,)),
            pltpu.SemaphoreType.DMA((2,)),
        ],
        compiler_params=pltpu.CompilerParams(
            use_tc_tiling_on_sc=False, needs_layout_passes=False),
    )(_emb_body)

    out = emb(xt, pos_lin, table)
    return (out.reshape(S * 8, NBT, 8, 128)
            .transpose(0, 2, 1, 3)
            .reshape(S, D, B)
            .transpose(2, 0, 1))


# speedup vs baseline: 2.0501x; 1.7620x over previous
import functools

import jax
import jax.numpy as jnp
from jax import lax
from jax.experimental import pallas as pl
from jax.experimental.pallas import tpu as pltpu
from jax.experimental.pallas import tpu_sc as plsc

NUM_CORES = 2
NUM_SUBCORES = 16
NUM_WORKERS = NUM_CORES * NUM_SUBCORES
LANES = 16

B = 4096
S = 200
D = 64
BBLK = 256
ITEMS_PER_S = B // BBLK
N_ITEMS = S * ITEMS_PER_S
PER_W = N_ITEMS // NUM_WORKERS
CHUNK_W = 1024
CHUNKS = 16
ITEM_W = CHUNK_W * CHUNKS
NBT = B // 128


def _emb_body(x_hbm, pos_hbm, table_hbm, out_hbm,
              idx_v, rows_v, pos_v, t1_v, g_sem, p_sem, o_sem):
    wid = lax.axis_index("s") * NUM_CORES + lax.axis_index("c")
    gbase = wid * PER_W
    iota = lax.iota(jnp.int32, LANES)
    p257 = iota * 257

    def item_coords(g):
        s = g // ITEMS_PER_S
        bt2 = g % ITEMS_PER_S
        return s, bt2 * BBLK

    def chunk_off(g, j):
        s, b0 = item_coords(g)
        return ((s * 8 + j // 2) * NBT + (b0 // 128 + j % 2)) * CHUNK_W

    def start_fetch(g, bslot):
        s, b0 = item_coords(g)
        pltpu.sync_copy(x_hbm.at[s, pl.ds(b0, BBLK)], idx_v.at[bslot])
        pltpu.async_copy(table_hbm.at[idx_v.at[bslot]], rows_v.at[bslot],
                         g_sem.at[bslot])
        for j in range(CHUNKS):
            pltpu.async_copy(pos_hbm.at[pl.ds(chunk_off(g, j), CHUNK_W)],
                             pos_v.at[bslot, pl.ds(j * CHUNK_W, CHUNK_W)],
                             p_sem.at[bslot])

    def wait_fetch(bslot):
        pltpu.make_async_copy(table_hbm.at[idx_v.at[bslot]],
                              rows_v.at[bslot], g_sem.at[bslot]).wait()
        pltpu.make_async_copy(pos_hbm.at[pl.ds(0, ITEM_W)],
                              pos_v.at[bslot], p_sem.at[bslot]).wait()

    def wait_out(bslot):
        pltpu.make_async_copy(pos_v.at[bslot], out_hbm.at[pl.ds(0, ITEM_W)],
                              o_sem.at[bslot]).wait()

    start_fetch(gbase, 0)

    def pair_body(p, carry):
        for bslot in range(2):
            g = gbase + 2 * p + bslot
            wait_fetch(bslot)
            nxt = 1 - bslot

            @pl.when(2 * p + bslot + 1 < PER_W)
            def _():
                @pl.when(2 * p + bslot >= 1)
                def _():
                    wait_out(nxt)

                start_fetch(g + 1, nxt)

            def b_body(b, c):
                vals = [rows_v[bslot, b, pl.ds(k * LANES, LANES)]
                        for k in range(D // LANES)]
                for k in range(D // LANES):
                    plsc.store_scatter(t1_v, [p257 + (k * LANES * 257 + b)],
                                       vals[k])
                return c

            lax.fori_loop(0, BBLK, b_body, 0, unroll=4)

            def d_body(d, c):
                tbase = d * 257
                pbase = (d // 8) * 2048 + (d % 8) * 128
                for jj in range(4):
                    vals = []
                    for j in range(4):
                        b0 = (jj * 4 + j) * LANES
                        vals.append(
                            plsc.load_gather(t1_v, [iota + (tbase + b0)]))
                    for j in range(4):
                        b0 = (jj * 4 + j) * LANES
                        po = pbase + (b0 // 128) * 1024 + (b0 % 128)
                        plsc.addupdate(pos_v.at[bslot, pl.ds(po, LANES)],
                                       vals[j])
                return c

            lax.fori_loop(0, D, d_body, 0, unroll=2)
            for j in range(CHUNKS):
                pltpu.async_copy(pos_v.at[bslot, pl.ds(j * CHUNK_W, CHUNK_W)],
                                 out_hbm.at[pl.ds(chunk_off(g, j), CHUNK_W)],
                                 o_sem.at[bslot])
        return carry

    lax.fori_loop(0, PER_W // 2, pair_body, 0)
    wait_out(0)
    wait_out(1)


def kernel(x, positional_embedding, table):
    xt = x.T
    pos_lin = (positional_embedding.transpose(1, 2, 0)
               .reshape(S * 8, 8, NBT, 128)
               .transpose(0, 2, 1, 3)
               .reshape(-1))

    mesh = plsc.VectorSubcoreMesh(
        core_axis_name="c", subcore_axis_name="s",
        num_cores=NUM_CORES, num_subcores=NUM_SUBCORES)

    emb = functools.partial(
        pl.kernel,
        out_type=jax.ShapeDtypeStruct((B * S * D,), jnp.float32),
        mesh=mesh,
        scratch_types=[
            pltpu.VMEM((2, BBLK), jnp.int32),
            pltpu.VMEM((2, BBLK, D), jnp.float32),
            pltpu.VMEM((2, ITEM_W), jnp.float32),
            pltpu.VMEM((D * 257,), jnp.float32),
            pltpu.SemaphoreType.DMA((2,)),
            pltpu.SemaphoreType.DMA((2,)),
            pltpu.SemaphoreType.DMA((2,)),
        ],
        compiler_params=pltpu.CompilerParams(
            use_tc_tiling_on_sc=False, needs_layout_passes=False),
    )(_emb_body)

    out = emb(xt, pos_lin, table)
    return (out.reshape(S * 8, NBT, 8, 128)
            .transpose(0, 2, 1, 3)
            .reshape(S, D, B)
            .transpose(2, 0, 1))
